# Initial kernel scaffold; baseline (speedup 1.0000x reference)
import jax
import jax.numpy as jnp
from jax.experimental import pallas as pl


def kernel(tokens, charges, aa_table, charge_table):
    raise NotImplementedError("write your pallas kernel here")


# speedup vs baseline: 1.8373x; 1.8373x over previous
import functools

import jax
import jax.numpy as jnp
import numpy as np
from jax import lax
from jax.experimental import pallas as pl
from jax.experimental.pallas import tpu as pltpu
from jax.experimental.pallas import tpu_sc as plsc

B, L, DIM = 4096, 50, 512
VOCAB = 22
MAX_CHARGE = 10

NC, NS = 2, 16
NW = NC * NS
BPW = B // NW
NSLOT = 3
TROWS = 1104
RPS = TROWS // NS
TOKPAD = 128
NCHUNK = DIM // 16


def _pos_encoding() -> jnp.ndarray:
    pos = np.arange(L, dtype=np.float64)[:, None]
    i = np.arange(DIM // 2, dtype=np.float64)[None, :]
    angle = pos / (10000.0 ** (2.0 * i / DIM))
    pe = np.zeros((L, DIM), dtype=np.float32)
    pe[:, 0::2] = np.sin(angle)
    pe[:, 1::2] = np.cos(angle)
    return jnp.asarray(pe)


def _body(tokens_hbm, charges_hbm, aa_hbm, pe_hbm, ct_hbm, out_hbm,
          buf0, buf1, buf2, idx0, idx1, idx2,
          tokens_v, charges_v, ct_v, t_sh,
          g0, g1, g2, w0, w1, w2):
    bufs = (buf0, buf1, buf2)
    idxs = (idx0, idx1, idx2)
    gsems = (g0, g1, g2)
    wsems = (w0, w1, w2)

    s = lax.axis_index("s")
    c = lax.axis_index("c")
    wid = s * NC + c

    pltpu.sync_copy(aa_hbm, buf0.at[pl.ds(0, VOCAB)])
    pltpu.sync_copy(pe_hbm, buf1)

    def build_row(k, carry):
        r = s * RPS + k
        v = r // L
        l = r - v * L
        for j in range(NCHUNK):
            buf2[0, pl.ds(16 * j, 16)] = (
                buf0[v, pl.ds(16 * j, 16)] + buf1[l, pl.ds(16 * j, 16)])
        pltpu.sync_copy(buf2.at[pl.ds(0, 1)], t_sh.at[pl.ds(r, 1)])
        return carry

    lax.fori_loop(0, RPS, build_row, 0)

    tok_base = wid * (BPW * L)
    pltpu.sync_copy(tokens_hbm.at[pl.ds(tok_base, BPW * L)],
                    tokens_v.at[pl.ds(0, BPW * L)])
    zeros16 = jnp.zeros((16,), jnp.int32)
    for m in range(TOKPAD // 16):
        tokens_v[pl.ds(BPW * L + 16 * m, 16)] = zeros16
    pltpu.sync_copy(charges_hbm.at[pl.ds(wid * BPW, BPW)],
                    charges_v.at[pl.ds(0, BPW)])
    pltpu.sync_copy(ct_hbm, ct_v)

    plsc.subcore_barrier()

    iotas = [lax.iota(jnp.int32, 16) + (16 * k) for k in range(4)]

    def comp_idx(i, slot):
        base = i * L
        for k in range(4):
            tv = tokens_v[pl.ds(base + 16 * k, 16)]
            idxs[slot][pl.ds(16 * k, 16)] = tv * L + iotas[k]

    def start_gather(slot):
        return pltpu.make_async_copy(
            t_sh.at[idxs[slot].at[pl.ds(0, L)]], bufs[slot], gsems[slot])

    def start_write(i, slot):
        return pltpu.make_async_copy(
            bufs[slot], out_hbm.at[pl.ds(wid * (BPW * L) + i * L, L)],
            wsems[slot])

    def charge_add(i, slot):
        cc = charges_v[pl.ds(i, 16)][0]
        cvs = [ct_v[cc, pl.ds(16 * j, 16)] for j in range(NCHUNK)]

        def row_add(r, rc, _buf=bufs[slot], _cvs=cvs):
            for j in range(NCHUNK):
                plsc.addupdate(_buf.at[r, pl.ds(16 * j, 16)], _cvs[j])
            return rc

        lax.fori_loop(0, L, row_add, 0)

    comp_idx(0, 0)
    start_gather(0).start()

    NFULL = (BPW // NSLOT) * NSLOT

    def gbody(g, carry):
        for b in range(NSLOT):
            i = g * NSLOT + b
            slot = b
            s2 = (b + 1) % NSLOT
            comp_idx(i + 1, s2)
            if b == NSLOT - 1:
                start_write(i, s2).wait()
            else:
                @pl.when(g > 0)
                def _():
                    start_write(i, s2).wait()
            start_gather(s2).start()
            start_gather(slot).wait()
            charge_add(i, slot)
            start_write(i, slot).start()
        return carry

    lax.fori_loop(0, NFULL // NSLOT, gbody, 0)

    for i in range(NFULL, BPW):
        slot = i % NSLOT
        s2 = (i + 1) % NSLOT
        if i + 1 < BPW:
            comp_idx(i + 1, s2)
            start_write(i, s2).wait()
            start_gather(s2).start()
        start_gather(slot).wait()
        charge_add(i, slot)
        start_write(i, slot).start()

    for i in range(BPW - NSLOT, BPW):
        start_write(0, i % NSLOT).wait()


@functools.partial(jax.jit, static_argnames=())
def _run(tokens_flat, charges, aa_table, pe, charge_table):
    mesh = plsc.VectorSubcoreMesh(core_axis_name="c", subcore_axis_name="s")
    fn = pl.kernel(
        _body,
        mesh=mesh,
        compiler_params=pltpu.CompilerParams(use_tc_tiling_on_sc=False),
        out_type=jax.ShapeDtypeStruct((B * L, DIM), jnp.float32),
        scratch_types=(
            [pltpu.VMEM((L, DIM), jnp.float32) for _ in range(NSLOT)]
            + [pltpu.VMEM((64,), jnp.int32) for _ in range(NSLOT)]
            + [pltpu.VMEM((BPW * L + TOKPAD,), jnp.int32),
               pltpu.VMEM((BPW + 16,), jnp.int32),
               pltpu.VMEM((MAX_CHARGE, DIM), jnp.float32),
               pltpu.VMEM_SHARED((TROWS, DIM), jnp.float32)]
            + [pltpu.SemaphoreType.DMA for _ in range(2 * NSLOT)]),
    )
    return fn(tokens_flat, charges, aa_table, pe, charge_table)


def kernel(tokens, charges, aa_table, charge_table):
    aa = aa_table.at[0].set(0.0)
    tokens_flat = tokens.astype(jnp.int32).reshape(-1)
    out = _run(tokens_flat, charges.astype(jnp.int32), aa,
               _pos_encoding(), charge_table)
    return out.reshape(B, L, DIM)


# speedup vs baseline: 6.5888x; 3.5861x over previous
import functools

import jax
import jax.numpy as jnp
import numpy as np
from jax import lax
from jax.experimental import pallas as pl
from jax.experimental.pallas import tpu as pltpu
from jax.experimental.pallas import tpu_sc as plsc

B, L, DIM = 4096, 50, 512
VOCAB = 22
MAX_CHARGE = 10

NC, NS = 2, 16
NW = NC * NS
BPW = B // NW
NSLOT = 3
TROWS = 1104
RPS = TROWS // NS
TOKPAD = 128
NCHUNK = DIM // 16


def _pos_encoding() -> jnp.ndarray:
    pos = np.arange(L, dtype=np.float64)[:, None]
    i = np.arange(DIM // 2, dtype=np.float64)[None, :]
    angle = pos / (10000.0 ** (2.0 * i / DIM))
    pe = np.zeros((L, DIM), dtype=np.float32)
    pe[:, 0::2] = np.sin(angle)
    pe[:, 1::2] = np.cos(angle)
    return jnp.asarray(pe)


def _body(tokens_hbm, charges_hbm, aa_hbm, pe_hbm, ct_hbm, out_hbm,
          buf0, buf1, buf2, idx0, idx1, idx2,
          tokens_v, charges_v, ct_v, t_sh,
          g0, g1, g2, w0, w1, w2):
    bufs = (buf0, buf1, buf2)
    idxs = (idx0, idx1, idx2)
    gsems = (g0, g1, g2)
    wsems = (w0, w1, w2)

    s = lax.axis_index("s")
    c = lax.axis_index("c")
    wid = s * NC + c

    pltpu.sync_copy(aa_hbm, buf0.at[pl.ds(0, VOCAB)])
    pltpu.sync_copy(pe_hbm, buf1)

    def build_row(k, carry):
        r = s * RPS + k
        v = r // L
        l = r - v * L
        for j in range(NCHUNK):
            tc, dm = j // 8, (16 * j) % 128
            buf2[0, tc, pl.ds(dm, 16)] = (
                buf0[v, tc, pl.ds(dm, 16)] + buf1[l, tc, pl.ds(dm, 16)])
        pltpu.sync_copy(buf2.at[pl.ds(0, 1)], t_sh.at[pl.ds(r, 1)])
        return carry

    lax.fori_loop(0, RPS, build_row, 0)

    tok_base = wid * (BPW * L)
    pltpu.sync_copy(tokens_hbm.at[pl.ds(tok_base, BPW * L)],
                    tokens_v.at[pl.ds(0, BPW * L)])
    zeros16 = jnp.zeros((16,), jnp.int32)
    for m in range(TOKPAD // 16):
        tokens_v[pl.ds(BPW * L + 16 * m, 16)] = zeros16
    pltpu.sync_copy(charges_hbm.at[pl.ds(wid * BPW, BPW)],
                    charges_v.at[pl.ds(0, BPW)])
    pltpu.sync_copy(ct_hbm, ct_v)

    plsc.subcore_barrier()

    iotas = [lax.iota(jnp.int32, 16) + (16 * k) for k in range(4)]

    def comp_idx(i, slot):
        base = i * L
        for k in range(4):
            tv = tokens_v[pl.ds(base + 16 * k, 16)]
            idxs[slot][pl.ds(16 * k, 16)] = tv * L + iotas[k]

    def start_gather(slot):
        return pltpu.make_async_copy(
            t_sh.at[idxs[slot].at[pl.ds(0, L)]], bufs[slot], gsems[slot])

    def start_write(i, slot):
        gb = wid * BPW + i
        return pltpu.make_async_copy(
            bufs[slot], out_hbm.at[:, gb // 8, :, gb % 8, :], wsems[slot])

    def charge_add(i, slot):
        cc = charges_v[pl.ds(i, 16)][0]
        cvs = [ct_v[cc, pl.ds(16 * j, 16)] for j in range(NCHUNK)]

        def row_add(r, rc, _buf=bufs[slot], _cvs=cvs):
            for j in range(NCHUNK):
                plsc.addupdate(
                    _buf.at[r, j // 8, pl.ds((16 * j) % 128, 16)], _cvs[j])
            return rc

        lax.fori_loop(0, L, row_add, 0)

    comp_idx(0, 0)
    start_gather(0).start()

    NFULL = (BPW // NSLOT) * NSLOT

    def gbody(g, carry):
        for b in range(NSLOT):
            i = g * NSLOT + b
            slot = b
            s2 = (b + 1) % NSLOT
            comp_idx(i + 1, s2)
            if b == NSLOT - 1:
                start_write(i, s2).wait()
            else:
                @pl.when(g > 0)
                def _():
                    start_write(i, s2).wait()
            start_gather(s2).start()
            start_gather(slot).wait()
            charge_add(i, slot)
            start_write(i, slot).start()
        return carry

    lax.fori_loop(0, NFULL // NSLOT, gbody, 0)

    for i in range(NFULL, BPW):
        slot = i % NSLOT
        s2 = (i + 1) % NSLOT
        if i + 1 < BPW:
            comp_idx(i + 1, s2)
            start_write(i, s2).wait()
            start_gather(s2).start()
        start_gather(slot).wait()
        charge_add(i, slot)
        start_write(i, slot).start()

    for i in range(BPW - NSLOT, BPW):
        start_write(0, i % NSLOT).wait()


@functools.partial(jax.jit, static_argnames=())
def _run(tokens_flat, charges, aa_table, pe, charge_table):
    mesh = plsc.VectorSubcoreMesh(core_axis_name="c", subcore_axis_name="s")
    fn = pl.kernel(
        _body,
        mesh=mesh,
        compiler_params=pltpu.CompilerParams(use_tc_tiling_on_sc=False),
        out_type=jax.ShapeDtypeStruct((L, B // 8, DIM // 128, 8, 128),
                                      jnp.float32),
        scratch_types=(
            [pltpu.VMEM((L, DIM // 128, 128), jnp.float32)
             for _ in range(NSLOT)]
            + [pltpu.VMEM((64,), jnp.int32) for _ in range(NSLOT)]
            + [pltpu.VMEM((BPW * L + TOKPAD,), jnp.int32),
               pltpu.VMEM((BPW + 16,), jnp.int32),
               pltpu.VMEM((MAX_CHARGE, DIM), jnp.float32),
               pltpu.VMEM_SHARED((TROWS, DIM // 128, 128), jnp.float32)]
            + [pltpu.SemaphoreType.DMA for _ in range(2 * NSLOT)]),
    )
    return fn(tokens_flat, charges, aa_table, pe, charge_table)


def kernel(tokens, charges, aa_table, charge_table):
    aa = aa_table.at[0].set(0.0).reshape(VOCAB, DIM // 128, 128)
    pe = _pos_encoding().reshape(L, DIM // 128, 128)
    tokens_flat = tokens.astype(jnp.int32).reshape(-1)
    out5 = _run(tokens_flat, charges.astype(jnp.int32), aa, pe, charge_table)
    return jnp.transpose(out5, (1, 3, 0, 2, 4)).reshape(B, L, DIM)
